# baseline (speedup 1.0000x reference)
import itertools

import jax
import jax.numpy as jnp
import numpy as np
from jax.experimental import pallas as pl
from jax.experimental.pallas import tpu as pltpu

BS = 4
H, W = 480, 640
NC = 16
C = 4 + NC
RES = 5
ZRES = 5
MAP_CM = 4800 // 2
M = MAP_CM // RES
VR = 100
FOV = 79.0
DU = 1
AGENT_H = 1.55 * 100.0
MAXH = int(360 / ZRES)
MINH = int(-40 / ZRES)
ZH = MAXH - MINH
MAP_THR, EXP_THR, CAT_THR = 1.0, 1.0, 5.0
DEG = 57.29577951308232
MIN_Z = int(5 / ZRES - MINH)
MAX_Z = int((AGENT_H + 1 + 50) / ZRES - MINH)
XCAM = (W - 1.0) / 2.0
ZCAM = (H - 1.0) / 2.0
FOC = (W / 2.0) / np.tan(np.deg2rad(FOV / 2.0))

NF = 1 + NC
G = VR * VR * ZH
XY = VR * VR
CHUNK = 200
NCH = XY // CHUNK
NOUT = 2 + NC


def _splat_corner_grids(feat, coords):
    B = feat.shape[0]
    grid_dims = (VR, VR, ZH)
    pos_dim, wts_dim = [], []
    for d in range(3):
        gd = grid_dims[d]
        pos = coords[:, d, :] * (gd / 2.0) + gd / 2.0
        fl = jnp.floor(pos)
        pd, wd = [], []
        for ix in (0.0, 1.0):
            pos_ix = fl + ix
            safe = ((pos_ix > 0) & (pos_ix < gd)).astype(feat.dtype)
            wd.append((1.0 - jnp.abs(pos - pos_ix)) * safe)
            pd.append(pos_ix * safe)
        pos_dim.append(pd)
        wts_dim.append(wd)
    grids = []
    for ix_d in itertools.product((0, 1), (0, 1), (0, 1)):
        w = wts_dim[0][ix_d[0]] * wts_dim[1][ix_d[1]] * wts_dim[2][ix_d[2]]
        idx = jnp.zeros_like(w)
        for d in range(3):
            idx = idx * grid_dims[d] + pos_dim[d][ix_d[d]]
        idx = idx.astype(jnp.int32)
        vals = feat * w[:, None, :]
        g = jax.vmap(
            lambda i, v: jnp.zeros((NF, G), jnp.float32).at[:, i].add(v)
        )(idx, vals)
        grids.append(g.reshape(B, NF, NCH, CHUNK, ZH))
    return grids


def _merge_kernel(g0, g1, g2, g3, g4, g5, g6, g7, o_ref):
    s = (((g0[0, :, 0] + g1[0, :, 0]) + (g2[0, :, 0] + g3[0, :, 0]))
         + ((g4[0, :, 0] + g5[0, :, 0]) + (g6[0, :, 0] + g7[0, :, 0])))
    s = jnp.round(s)
    band = jnp.sum(s[..., MIN_Z:MAX_Z], axis=-1)
    full0 = jnp.sum(s[0:1], axis=-1)
    fp_map = jnp.clip(band[0:1] / MAP_THR, 0.0, 1.0)
    fp_exp = jnp.clip(full0 / EXP_THR, 0.0, 1.0)
    cat = jnp.clip(band[1:] / CAT_THR, 0.0, 1.0)
    o_ref[0, 0] = jnp.concatenate([fp_map, fp_exp, cat], axis=0)


def _merge_grids(grids):
    B = grids[0].shape[0]
    spec = pl.BlockSpec((1, NF, 1, CHUNK, ZH), lambda b, j: (b, 0, j, 0, 0))
    out = pl.pallas_call(
        _merge_kernel,
        out_shape=jax.ShapeDtypeStruct((B, NCH, NOUT, CHUNK), jnp.float32),
        grid=(B, NCH),
        in_specs=[spec] * 8,
        out_specs=pl.BlockSpec((1, 1, NOUT, CHUNK), lambda b, j: (b, j, 0, 0)),
        compiler_params=pltpu.CompilerParams(
            dimension_semantics=("parallel", "arbitrary"),
        ),
    )(*grids)
    return out.swapaxes(1, 2).reshape(B, NOUT, XY)


def _affine_grid(theta, Hh, Ww):
    xs = jnp.linspace(-1.0, 1.0, Ww)
    ys = jnp.linspace(-1.0, 1.0, Hh)
    Xg, Yg = jnp.meshgrid(xs, ys)
    base = jnp.stack([Xg, Yg, jnp.ones_like(Xg)], -1)
    return jnp.einsum('bij,hwj->bhwi', theta, base)


def _grid_sample(img, grid):
    B, Cc, Hh, Ww = img.shape
    x = (grid[..., 0] + 1.0) * 0.5 * (Ww - 1)
    y = (grid[..., 1] + 1.0) * 0.5 * (Hh - 1)
    x0 = jnp.floor(x)
    y0 = jnp.floor(y)
    wx1 = x - x0
    wy1 = y - y0

    def gather(ix, iy):
        valid = ((ix >= 0) & (ix <= Ww - 1) & (iy >= 0) & (iy <= Hh - 1)).astype(img.dtype)
        ixc = jnp.clip(ix, 0, Ww - 1).astype(jnp.int32)
        iyc = jnp.clip(iy, 0, Hh - 1).astype(jnp.int32)
        v = jax.vmap(lambda im, yy, xx: im[:, yy, xx])(img, iyc, ixc)
        return v * valid[:, None]

    return (gather(x0, y0) * ((1 - wx1) * (1 - wy1))[:, None]
            + gather(x0 + 1, y0) * (wx1 * (1 - wy1))[:, None]
            + gather(x0, y0 + 1) * ((1 - wx1) * wy1)[:, None]
            + gather(x0 + 1, y0 + 1) * (wx1 * wy1)[:, None])


def _max_kernel(a_ref, b_ref, o_ref):
    o_ref[...] = jnp.maximum(a_ref[...], b_ref[...])


def _pallas_max(a, b):
    return pl.pallas_call(
        _max_kernel,
        out_shape=jax.ShapeDtypeStruct(a.shape, a.dtype),
        grid=(a.shape[0], a.shape[1] // 4),
        in_specs=[
            pl.BlockSpec((1, 4, M, M), lambda i, j: (i, j, 0, 0)),
            pl.BlockSpec((1, 4, M, M), lambda i, j: (i, j, 0, 0)),
        ],
        out_specs=pl.BlockSpec((1, 4, M, M), lambda i, j: (i, j, 0, 0)),
        compiler_params=pltpu.CompilerParams(
            dimension_semantics=("parallel", "arbitrary"),
        ),
    )(a, b)


def kernel(obs, pose_obs, maps_last, poses_last, view_angles):
    bs = obs.shape[0]
    depth = obs[:, 3, ::DU, ::DU]
    gx = jnp.arange(W, dtype=obs.dtype)[None, None, ::DU]
    gz = jnp.arange(H - 1, -1, -1, dtype=obs.dtype)[None, ::DU, None]
    Xp = (gx - XCAM) * depth / FOC
    Zp = (gz - ZCAM) * depth / FOC
    a = jnp.deg2rad(view_angles)[:, None, None]
    ca, sa = jnp.cos(a), jnp.sin(a)
    Xv = Xp
    Yv = ca * depth - sa * Zp
    Zv = sa * depth + ca * Zp + AGENT_H
    Xv = Xv + VR * RES / 2.0
    xs = (Xv / RES - VR // 2.0) / VR * 2.0
    ys = (Yv / RES - VR // 2.0) / VR * 2.0
    zs = (Zv / ZRES - (MAXH + MINH) // 2.0) / (MAXH - MINH) * 2.0
    coords = jnp.stack([xs, ys, zs], 1).reshape(bs, 3, -1)
    sem = obs[:, 4:]
    pooled = sem.reshape(bs, NC, H // DU, DU, W // DU, DU).mean((3, 5))
    N = (H // DU) * (W // DU)
    feat = jnp.concatenate([jnp.ones((bs, 1, N), obs.dtype), pooled.reshape(bs, NC, N)], 1)

    grids = _splat_corner_grids(feat, coords)
    maps_xy = _merge_grids(grids)
    maps2d = maps_xy.reshape(bs, NOUT, VR, VR).swapaxes(2, 3)
    fp_map_pred = maps2d[:, 0:1]

    agent_view = jnp.zeros((bs, C, M, M), obs.dtype)
    x1 = M // 2 - VR // 2
    x2 = x1 + VR
    y1 = M // 2
    y2 = y1 + VR
    agent_view = agent_view.at[:, 0:1, y1:y2, x1:x2].set(fp_map_pred)
    agent_view = agent_view.at[:, 1:2, y1:y2, x1:x2].set(maps2d[:, 1:2])
    agent_view = agent_view.at[:, 4:, y1:y2, x1:x2].set(maps2d[:, 2:])

    o = poses_last[:, 2] / DEG
    yy = poses_last[:, 1] + pose_obs[:, 0] * jnp.sin(o) + pose_obs[:, 1] * jnp.cos(o)
    xx = poses_last[:, 0] + pose_obs[:, 0] * jnp.cos(o) - pose_obs[:, 1] * jnp.sin(o)
    tt = poses_last[:, 2] + pose_obs[:, 2] * DEG
    tt = jnp.fmod(tt - 180.0, 360.0) + 180.0
    tt = jnp.fmod(tt + 180.0, 360.0) - 180.0
    current_poses = jnp.stack([xx, yy, tt], 1)
    st = jax.lax.stop_gradient(current_poses)
    half = M // 2
    stx = -(st[:, 0] * 100.0 / RES - half) / half
    sty = -(st[:, 1] * 100.0 / RES - half) / half
    t = (90.0 - st[:, 2]) * np.pi / 180.0
    ct, s_t = jnp.cos(t), jnp.sin(t)
    zero, one = jnp.zeros_like(ct), jnp.ones_like(ct)
    theta1 = jnp.stack([jnp.stack([ct, -s_t, zero], 1), jnp.stack([s_t, ct, zero], 1)], 1)
    theta2 = jnp.stack([jnp.stack([one, zero, stx], 1), jnp.stack([zero, one, sty], 1)], 1)
    rotated = _grid_sample(agent_view, _affine_grid(theta1, M, M))
    translated = _grid_sample(rotated, _affine_grid(theta2, M, M))
    map_pred = _pallas_max(maps_last, translated)
    return fp_map_pred, map_pred, current_poses, current_poses, translated


# speedup vs baseline: 1.1744x; 1.1744x over previous
import itertools

import jax
import jax.numpy as jnp
import numpy as np
from jax.experimental import pallas as pl
from jax.experimental.pallas import tpu as pltpu

BS = 4
H, W = 480, 640
NC = 16
C = 4 + NC
RES = 5
ZRES = 5
MAP_CM = 4800 // 2
M = MAP_CM // RES
VR = 100
FOV = 79.0
DU = 1
AGENT_H = 1.55 * 100.0
MAXH = int(360 / ZRES)
MINH = int(-40 / ZRES)
ZH = MAXH - MINH
MAP_THR, EXP_THR, CAT_THR = 1.0, 1.0, 5.0
DEG = 57.29577951308232
MIN_Z = int(5 / ZRES - MINH)
MAX_Z = int((AGENT_H + 1 + 50) / ZRES - MINH)
XCAM = (W - 1.0) / 2.0
ZCAM = (H - 1.0) / 2.0
FOC = (W / 2.0) / np.tan(np.deg2rad(FOV / 2.0))

NF = 1 + NC
G = VR * VR * ZH
XY = VR * VR
CHUNK = 200
NCH = XY // CHUNK
NOUT = 2 + NC


def _splat_corner_grids(feat, coords):
    B = feat.shape[0]
    grid_dims = (VR, VR, ZH)
    pos_dim, wts_dim = [], []
    for d in range(3):
        gd = grid_dims[d]
        pos = coords[:, d, :] * (gd / 2.0) + gd / 2.0
        fl = jnp.floor(pos)
        pd, wd = [], []
        for ix in (0.0, 1.0):
            pos_ix = fl + ix
            safe = ((pos_ix > 0) & (pos_ix < gd)).astype(feat.dtype)
            wd.append((1.0 - jnp.abs(pos - pos_ix)) * safe)
            pd.append(pos_ix * safe)
        pos_dim.append(pd)
        wts_dim.append(wd)
    grids = [None, None]
    for ci, ix_d in enumerate(itertools.product((0, 1), (0, 1), (0, 1))):
        w = wts_dim[0][ix_d[0]] * wts_dim[1][ix_d[1]] * wts_dim[2][ix_d[2]]
        idx = jnp.zeros_like(w)
        for d in range(3):
            idx = idx * grid_dims[d] + pos_dim[d][ix_d[d]]
        idx = idx.astype(jnp.int32)
        vals = feat * w[:, None, :]
        half = ci % 2
        if grids[half] is None:
            grids[half] = jax.vmap(
                lambda i, v: jnp.zeros((NF, G), jnp.float32).at[:, i].add(v)
            )(idx, vals)
        else:
            grids[half] = jax.vmap(
                lambda g, i, v: g.at[:, i].add(v)
            )(grids[half], idx, vals)
    return [g.reshape(B, NF, NCH, CHUNK, ZH) for g in grids]


def _merge_kernel(g0, g1, o_ref):
    s = g0[0, :, 0] + g1[0, :, 0]
    s = jnp.round(s)
    band = jnp.sum(s[..., MIN_Z:MAX_Z], axis=-1)
    full0 = jnp.sum(s[0:1], axis=-1)
    fp_map = jnp.clip(band[0:1] / MAP_THR, 0.0, 1.0)
    fp_exp = jnp.clip(full0 / EXP_THR, 0.0, 1.0)
    cat = jnp.clip(band[1:] / CAT_THR, 0.0, 1.0)
    o_ref[0, 0] = jnp.concatenate([fp_map, fp_exp, cat], axis=0)


def _merge_grids(grids):
    B = grids[0].shape[0]
    spec = pl.BlockSpec((1, NF, 1, CHUNK, ZH), lambda b, j: (b, 0, j, 0, 0))
    out = pl.pallas_call(
        _merge_kernel,
        out_shape=jax.ShapeDtypeStruct((B, NCH, NOUT, CHUNK), jnp.float32),
        grid=(B, NCH),
        in_specs=[spec] * len(grids),
        out_specs=pl.BlockSpec((1, 1, NOUT, CHUNK), lambda b, j: (b, j, 0, 0)),
        compiler_params=pltpu.CompilerParams(
            dimension_semantics=("parallel", "arbitrary"),
        ),
    )(*grids)
    return out.swapaxes(1, 2).reshape(B, NOUT, XY)


def _affine_grid(theta, Hh, Ww):
    xs = jnp.linspace(-1.0, 1.0, Ww)
    ys = jnp.linspace(-1.0, 1.0, Hh)
    Xg, Yg = jnp.meshgrid(xs, ys)
    base = jnp.stack([Xg, Yg, jnp.ones_like(Xg)], -1)
    return jnp.einsum('bij,hwj->bhwi', theta, base)


def _grid_sample(img, grid):
    B, Cc, Hh, Ww = img.shape
    x = (grid[..., 0] + 1.0) * 0.5 * (Ww - 1)
    y = (grid[..., 1] + 1.0) * 0.5 * (Hh - 1)
    x0 = jnp.floor(x)
    y0 = jnp.floor(y)
    wx1 = x - x0
    wy1 = y - y0

    def gather(ix, iy):
        valid = ((ix >= 0) & (ix <= Ww - 1) & (iy >= 0) & (iy <= Hh - 1)).astype(img.dtype)
        ixc = jnp.clip(ix, 0, Ww - 1).astype(jnp.int32)
        iyc = jnp.clip(iy, 0, Hh - 1).astype(jnp.int32)
        v = jax.vmap(lambda im, yy, xx: im[:, yy, xx])(img, iyc, ixc)
        return v * valid[:, None]

    return (gather(x0, y0) * ((1 - wx1) * (1 - wy1))[:, None]
            + gather(x0 + 1, y0) * (wx1 * (1 - wy1))[:, None]
            + gather(x0, y0 + 1) * ((1 - wx1) * wy1)[:, None]
            + gather(x0 + 1, y0 + 1) * (wx1 * wy1)[:, None])


def _max_kernel(a_ref, b_ref, o_ref):
    o_ref[...] = jnp.maximum(a_ref[...], b_ref[...])


def _pallas_max(a, b):
    return pl.pallas_call(
        _max_kernel,
        out_shape=jax.ShapeDtypeStruct(a.shape, a.dtype),
        grid=(a.shape[0], a.shape[1] // 4),
        in_specs=[
            pl.BlockSpec((1, 4, M, M), lambda i, j: (i, j, 0, 0)),
            pl.BlockSpec((1, 4, M, M), lambda i, j: (i, j, 0, 0)),
        ],
        out_specs=pl.BlockSpec((1, 4, M, M), lambda i, j: (i, j, 0, 0)),
        compiler_params=pltpu.CompilerParams(
            dimension_semantics=("parallel", "arbitrary"),
        ),
    )(a, b)


def kernel(obs, pose_obs, maps_last, poses_last, view_angles):
    bs = obs.shape[0]
    depth = obs[:, 3, ::DU, ::DU]
    gx = jnp.arange(W, dtype=obs.dtype)[None, None, ::DU]
    gz = jnp.arange(H - 1, -1, -1, dtype=obs.dtype)[None, ::DU, None]
    Xp = (gx - XCAM) * depth / FOC
    Zp = (gz - ZCAM) * depth / FOC
    a = jnp.deg2rad(view_angles)[:, None, None]
    ca, sa = jnp.cos(a), jnp.sin(a)
    Xv = Xp
    Yv = ca * depth - sa * Zp
    Zv = sa * depth + ca * Zp + AGENT_H
    Xv = Xv + VR * RES / 2.0
    xs = (Xv / RES - VR // 2.0) / VR * 2.0
    ys = (Yv / RES - VR // 2.0) / VR * 2.0
    zs = (Zv / ZRES - (MAXH + MINH) // 2.0) / (MAXH - MINH) * 2.0
    coords = jnp.stack([xs, ys, zs], 1).reshape(bs, 3, -1)
    sem = obs[:, 4:]
    pooled = sem.reshape(bs, NC, H // DU, DU, W // DU, DU).mean((3, 5))
    N = (H // DU) * (W // DU)
    feat = jnp.concatenate([jnp.ones((bs, 1, N), obs.dtype), pooled.reshape(bs, NC, N)], 1)

    grids = _splat_corner_grids(feat, coords)
    maps_xy = _merge_grids(grids)
    maps2d = maps_xy.reshape(bs, NOUT, VR, VR).swapaxes(2, 3)
    fp_map_pred = maps2d[:, 0:1]

    agent_view = jnp.zeros((bs, C, M, M), obs.dtype)
    x1 = M // 2 - VR // 2
    x2 = x1 + VR
    y1 = M // 2
    y2 = y1 + VR
    agent_view = agent_view.at[:, 0:1, y1:y2, x1:x2].set(fp_map_pred)
    agent_view = agent_view.at[:, 1:2, y1:y2, x1:x2].set(maps2d[:, 1:2])
    agent_view = agent_view.at[:, 4:, y1:y2, x1:x2].set(maps2d[:, 2:])

    o = poses_last[:, 2] / DEG
    yy = poses_last[:, 1] + pose_obs[:, 0] * jnp.sin(o) + pose_obs[:, 1] * jnp.cos(o)
    xx = poses_last[:, 0] + pose_obs[:, 0] * jnp.cos(o) - pose_obs[:, 1] * jnp.sin(o)
    tt = poses_last[:, 2] + pose_obs[:, 2] * DEG
    tt = jnp.fmod(tt - 180.0, 360.0) + 180.0
    tt = jnp.fmod(tt + 180.0, 360.0) - 180.0
    current_poses = jnp.stack([xx, yy, tt], 1)
    st = jax.lax.stop_gradient(current_poses)
    half = M // 2
    stx = -(st[:, 0] * 100.0 / RES - half) / half
    sty = -(st[:, 1] * 100.0 / RES - half) / half
    t = (90.0 - st[:, 2]) * np.pi / 180.0
    ct, s_t = jnp.cos(t), jnp.sin(t)
    zero, one = jnp.zeros_like(ct), jnp.ones_like(ct)
    theta1 = jnp.stack([jnp.stack([ct, -s_t, zero], 1), jnp.stack([s_t, ct, zero], 1)], 1)
    theta2 = jnp.stack([jnp.stack([one, zero, stx], 1), jnp.stack([zero, one, sty], 1)], 1)
    rotated = _grid_sample(agent_view, _affine_grid(theta1, M, M))
    translated = _grid_sample(rotated, _affine_grid(theta2, M, M))
    map_pred = _pallas_max(maps_last, translated)
    return fp_map_pred, map_pred, current_poses, current_poses, translated


# speedup vs baseline: 1.2826x; 1.0921x over previous
import itertools

import jax
import jax.numpy as jnp
import numpy as np
from jax.experimental import pallas as pl
from jax.experimental.pallas import tpu as pltpu

BS = 4
H, W = 480, 640
NC = 16
C = 4 + NC
RES = 5
ZRES = 5
MAP_CM = 4800 // 2
M = MAP_CM // RES
VR = 100
FOV = 79.0
DU = 1
AGENT_H = 1.55 * 100.0
MAXH = int(360 / ZRES)
MINH = int(-40 / ZRES)
ZH = MAXH - MINH
MAP_THR, EXP_THR, CAT_THR = 1.0, 1.0, 5.0
DEG = 57.29577951308232
MIN_Z = int(5 / ZRES - MINH)
MAX_Z = int((AGENT_H + 1 + 50) / ZRES - MINH)
XCAM = (W - 1.0) / 2.0
ZCAM = (H - 1.0) / 2.0
FOC = (W / 2.0) / np.tan(np.deg2rad(FOV / 2.0))

NF = 1 + NC
G = VR * VR * ZH
XY = VR * VR
CHUNK = 200
NCH = XY // CHUNK
NOUT = 2 + NC


def _splat_corner_grids(feat, coords):
    B = feat.shape[0]
    grid_dims = (VR, VR, ZH)
    pos_dim, wts_dim = [], []
    for d in range(3):
        gd = grid_dims[d]
        pos = coords[:, d, :] * (gd / 2.0) + gd / 2.0
        fl = jnp.floor(pos)
        pd, wd = [], []
        for ix in (0.0, 1.0):
            pos_ix = fl + ix
            safe = ((pos_ix > 0) & (pos_ix < gd)).astype(feat.dtype)
            wd.append((1.0 - jnp.abs(pos - pos_ix)) * safe)
            pd.append(pos_ix * safe)
        pos_dim.append(pd)
        wts_dim.append(wd)
    grid_flat = jnp.zeros((B, NF, G), jnp.float32)
    for ix_d in itertools.product((0, 1), (0, 1), (0, 1)):
        w = wts_dim[0][ix_d[0]] * wts_dim[1][ix_d[1]] * wts_dim[2][ix_d[2]]
        idx = jnp.zeros_like(w)
        for d in range(3):
            idx = idx * grid_dims[d] + pos_dim[d][ix_d[d]]
        idx = idx.astype(jnp.int32)
        vals = feat * w[:, None, :]
        grid_flat = jax.vmap(lambda g, i, v: g.at[:, i].add(v))(grid_flat, idx, vals)
    return [grid_flat.reshape(B, NF, NCH, CHUNK, ZH)]


def _merge_kernel(g0, o_ref):
    s = g0[0, :, 0]
    s = jnp.round(s)
    band = jnp.sum(s[..., MIN_Z:MAX_Z], axis=-1)
    full0 = jnp.sum(s[0:1], axis=-1)
    fp_map = jnp.clip(band[0:1] / MAP_THR, 0.0, 1.0)
    fp_exp = jnp.clip(full0 / EXP_THR, 0.0, 1.0)
    cat = jnp.clip(band[1:] / CAT_THR, 0.0, 1.0)
    o_ref[0, 0] = jnp.concatenate([fp_map, fp_exp, cat], axis=0)


def _merge_grids(grids):
    B = grids[0].shape[0]
    spec = pl.BlockSpec((1, NF, 1, CHUNK, ZH), lambda b, j: (b, 0, j, 0, 0))
    out = pl.pallas_call(
        _merge_kernel,
        out_shape=jax.ShapeDtypeStruct((B, NCH, NOUT, CHUNK), jnp.float32),
        grid=(B, NCH),
        in_specs=[spec] * len(grids),
        out_specs=pl.BlockSpec((1, 1, NOUT, CHUNK), lambda b, j: (b, j, 0, 0)),
        compiler_params=pltpu.CompilerParams(
            dimension_semantics=("parallel", "arbitrary"),
        ),
    )(*grids)
    return out.swapaxes(1, 2).reshape(B, NOUT, XY)


def _affine_grid(theta, Hh, Ww):
    xs = jnp.linspace(-1.0, 1.0, Ww)
    ys = jnp.linspace(-1.0, 1.0, Hh)
    Xg, Yg = jnp.meshgrid(xs, ys)
    base = jnp.stack([Xg, Yg, jnp.ones_like(Xg)], -1)
    return jnp.einsum('bij,hwj->bhwi', theta, base)


def _grid_sample(img, grid):
    B, Cc, Hh, Ww = img.shape
    x = (grid[..., 0] + 1.0) * 0.5 * (Ww - 1)
    y = (grid[..., 1] + 1.0) * 0.5 * (Hh - 1)
    x0 = jnp.floor(x)
    y0 = jnp.floor(y)
    wx1 = x - x0
    wy1 = y - y0

    def gather(ix, iy):
        valid = ((ix >= 0) & (ix <= Ww - 1) & (iy >= 0) & (iy <= Hh - 1)).astype(img.dtype)
        ixc = jnp.clip(ix, 0, Ww - 1).astype(jnp.int32)
        iyc = jnp.clip(iy, 0, Hh - 1).astype(jnp.int32)
        v = jax.vmap(lambda im, yy, xx: im[:, yy, xx])(img, iyc, ixc)
        return v * valid[:, None]

    return (gather(x0, y0) * ((1 - wx1) * (1 - wy1))[:, None]
            + gather(x0 + 1, y0) * (wx1 * (1 - wy1))[:, None]
            + gather(x0, y0 + 1) * ((1 - wx1) * wy1)[:, None]
            + gather(x0 + 1, y0 + 1) * (wx1 * wy1)[:, None])


def _max_kernel(a_ref, b_ref, o_ref):
    o_ref[...] = jnp.maximum(a_ref[...], b_ref[...])


def _pallas_max(a, b):
    return pl.pallas_call(
        _max_kernel,
        out_shape=jax.ShapeDtypeStruct(a.shape, a.dtype),
        grid=(a.shape[0], a.shape[1] // 4),
        in_specs=[
            pl.BlockSpec((1, 4, M, M), lambda i, j: (i, j, 0, 0)),
            pl.BlockSpec((1, 4, M, M), lambda i, j: (i, j, 0, 0)),
        ],
        out_specs=pl.BlockSpec((1, 4, M, M), lambda i, j: (i, j, 0, 0)),
        compiler_params=pltpu.CompilerParams(
            dimension_semantics=("parallel", "arbitrary"),
        ),
    )(a, b)


def kernel(obs, pose_obs, maps_last, poses_last, view_angles):
    bs = obs.shape[0]
    depth = obs[:, 3, ::DU, ::DU]
    gx = jnp.arange(W, dtype=obs.dtype)[None, None, ::DU]
    gz = jnp.arange(H - 1, -1, -1, dtype=obs.dtype)[None, ::DU, None]
    Xp = (gx - XCAM) * depth / FOC
    Zp = (gz - ZCAM) * depth / FOC
    a = jnp.deg2rad(view_angles)[:, None, None]
    ca, sa = jnp.cos(a), jnp.sin(a)
    Xv = Xp
    Yv = ca * depth - sa * Zp
    Zv = sa * depth + ca * Zp + AGENT_H
    Xv = Xv + VR * RES / 2.0
    xs = (Xv / RES - VR // 2.0) / VR * 2.0
    ys = (Yv / RES - VR // 2.0) / VR * 2.0
    zs = (Zv / ZRES - (MAXH + MINH) // 2.0) / (MAXH - MINH) * 2.0
    coords = jnp.stack([xs, ys, zs], 1).reshape(bs, 3, -1)
    sem = obs[:, 4:]
    pooled = sem.reshape(bs, NC, H // DU, DU, W // DU, DU).mean((3, 5))
    N = (H // DU) * (W // DU)
    feat = jnp.concatenate([jnp.ones((bs, 1, N), obs.dtype), pooled.reshape(bs, NC, N)], 1)

    grids = _splat_corner_grids(feat, coords)
    maps_xy = _merge_grids(grids)
    maps2d = maps_xy.reshape(bs, NOUT, VR, VR).swapaxes(2, 3)
    fp_map_pred = maps2d[:, 0:1]

    agent_view = jnp.zeros((bs, C, M, M), obs.dtype)
    x1 = M // 2 - VR // 2
    x2 = x1 + VR
    y1 = M // 2
    y2 = y1 + VR
    agent_view = agent_view.at[:, 0:1, y1:y2, x1:x2].set(fp_map_pred)
    agent_view = agent_view.at[:, 1:2, y1:y2, x1:x2].set(maps2d[:, 1:2])
    agent_view = agent_view.at[:, 4:, y1:y2, x1:x2].set(maps2d[:, 2:])

    o = poses_last[:, 2] / DEG
    yy = poses_last[:, 1] + pose_obs[:, 0] * jnp.sin(o) + pose_obs[:, 1] * jnp.cos(o)
    xx = poses_last[:, 0] + pose_obs[:, 0] * jnp.cos(o) - pose_obs[:, 1] * jnp.sin(o)
    tt = poses_last[:, 2] + pose_obs[:, 2] * DEG
    tt = jnp.fmod(tt - 180.0, 360.0) + 180.0
    tt = jnp.fmod(tt + 180.0, 360.0) - 180.0
    current_poses = jnp.stack([xx, yy, tt], 1)
    st = jax.lax.stop_gradient(current_poses)
    half = M // 2
    stx = -(st[:, 0] * 100.0 / RES - half) / half
    sty = -(st[:, 1] * 100.0 / RES - half) / half
    t = (90.0 - st[:, 2]) * np.pi / 180.0
    ct, s_t = jnp.cos(t), jnp.sin(t)
    zero, one = jnp.zeros_like(ct), jnp.ones_like(ct)
    theta1 = jnp.stack([jnp.stack([ct, -s_t, zero], 1), jnp.stack([s_t, ct, zero], 1)], 1)
    theta2 = jnp.stack([jnp.stack([one, zero, stx], 1), jnp.stack([zero, one, sty], 1)], 1)
    rotated = _grid_sample(agent_view, _affine_grid(theta1, M, M))
    translated = _grid_sample(rotated, _affine_grid(theta2, M, M))
    map_pred = _pallas_max(maps_last, translated)
    return fp_map_pred, map_pred, current_poses, current_poses, translated
